# baseline (speedup 1.0000x reference)
import functools

import jax
import jax.numpy as jnp
from jax import lax
from jax.experimental import pallas as pl
from jax.experimental.pallas import tpu as pltpu
from jax.experimental.pallas import tpu_sc as plsc

_D = 768
_TOP_K = 64
_TS = 2048
_L = 16
_NCH = _D // _L


def _sc_mask_kernel(pf_hbm, perm_hbm, out_hbm, pfv, permv, bitsv, maskv,
                    redv):
    b = lax.axis_index("s")

    @pl.when((lax.axis_index("c") == 0) & (b < 4))
    def _():
        pltpu.sync_copy(pf_hbm.at[b], pfv)
        pltpu.sync_copy(perm_hbm, permv)

        zero16f = jnp.zeros((_L,), jnp.float32)
        for i in range(_NCH):
            sl = pl.ds(i * _L, _L)
            maskv[sl] = zero16f
            bitsv[sl] = lax.bitcast_convert_type(jnp.abs(pfv[sl]),
                                                 jnp.int32)

        chunks = tuple(bitsv[pl.ds(c * _L, _L)] for c in range(_NCH))

        zero16i = jnp.zeros((_L,), jnp.int32)
        one16i = jnp.ones((_L,), jnp.int32)
        lane = lax.iota(jnp.int32, _L)

        def total_splat(vec):
            for k in (1, 2, 4, 8):
                vec = vec + vec.at[lane ^ k].get(mode="promise_in_bounds")
            return vec

        def count(pred):
            accs = [zero16i, zero16i, zero16i, zero16i]
            for c in range(_NCH):
                accs[c % 4] = accs[c % 4] + jnp.where(pred(c), one16i,
                                                      zero16i)
            return total_splat((accs[0] + accs[1]) + (accs[2] + accs[3]))

        def t_body(i, cur):
            cand = cur | (jnp.int32(1) << (jnp.int32(30) - i))
            cnt = count(lambda c: chunks[c] >= cand)
            return jnp.where(cnt >= _TOP_K, cand, cur)

        t = lax.fori_loop(0, 31, t_body,
                          jnp.zeros((_L,), jnp.int32))

        c_ge = count(lambda c: chunks[c] >= t)

        def idx_of(c):
            return lane + jnp.int32(c * _L)

        def count_sel(jcut):
            return count(lambda c: (chunks[c] > t)
                         | ((chunks[c] == t) & (idx_of(c) <= jcut)))

        def tie_search(_):
            def j_body(i, cur):
                cand = cur | (jnp.int32(1) << (jnp.int32(9) - i))
                return jnp.where(count_sel(cand) < _TOP_K, cand, cur)

            jmax = lax.fori_loop(0, 10, j_body, zero16i)
            return jnp.where(count_sel(zero16i) < _TOP_K, jmax + 1, 0)

        jcut = lax.cond(c_ge[0] == _TOP_K,
                        lambda _: jnp.full((_L,), _D, jnp.int32),
                        tie_search, 0)

        ones16f = jnp.ones((_L,), jnp.float32)
        for c in range(_NCH):
            sel = (chunks[c] > t) | ((chunks[c] == t)
                                     & (idx_of(c) <= jcut))
            plsc.store_scatter(maskv, [permv[pl.ds(c * _L, _L)]],
                               ones16f, mask=sel)

        pltpu.sync_copy(maskv, out_hbm.at[b])


def _sc_mask(pf, perm2):
    mesh = plsc.VectorSubcoreMesh(core_axis_name="c", subcore_axis_name="s")
    return pl.kernel(
        _sc_mask_kernel,
        out_type=jax.ShapeDtypeStruct((4, _D), jnp.float32),
        mesh=mesh,
        scratch_types=[
            pltpu.VMEM((_D,), jnp.float32),
            pltpu.VMEM((_D,), jnp.int32),
            pltpu.VMEM((_D,), jnp.int32),
            pltpu.VMEM((_D,), jnp.float32),
            pltpu.VMEM((_L,), jnp.int32),
        ],
        compiler_params=pltpu.CompilerParams(needs_layout_passes=False),
    )(pf, perm2)


def _tc_matmul_kernel(mask_ref, w_ref, nw_ref, x_ref, out_ref, weff_ref):
    b = pl.program_id(0)
    s = pl.program_id(1)

    @pl.when(s == 0)
    def _build_weff():
        weff_ref[...] = (w_ref[...]
                         + mask_ref[pl.ds(b, 1), :] * nw_ref[...]
                         ).astype(jnp.bfloat16)

    x_tile = x_ref[0].astype(jnp.bfloat16)
    out_ref[0] = jax.lax.dot_general(
        x_tile, weff_ref[...], (((1,), (1,)), ((), ())),
        preferred_element_type=jnp.float32)


def kernel(x, W, new_W, perm, prompt_boundary):
    B, S, D = x.shape
    pf = jax.lax.dynamic_index_in_dim(x, prompt_boundary, axis=1,
                                      keepdims=False)
    perm2 = perm.astype(jnp.int32)

    mask = _sc_mask(pf, perm2)

    grid = (B, S // _TS)
    return pl.pallas_call(
        _tc_matmul_kernel,
        grid=grid,
        in_specs=[
            pl.BlockSpec((B, D), lambda b, s: (0, 0)),
            pl.BlockSpec((D, D), lambda b, s: (0, 0)),
            pl.BlockSpec((D, D), lambda b, s: (0, 0)),
            pl.BlockSpec((1, _TS, D), lambda b, s: (b, s, 0)),
        ],
        out_specs=pl.BlockSpec((1, _TS, D), lambda b, s: (b, s, 0)),
        out_shape=jax.ShapeDtypeStruct((B, S, D), jnp.float32),
        scratch_shapes=[pltpu.VMEM((_D, _D), jnp.bfloat16)],
        compiler_params=pltpu.CompilerParams(
            dimension_semantics=("arbitrary", "arbitrary")),
    )(mask, W, new_W, x)


# speedup vs baseline: 1.0246x; 1.0246x over previous
import functools

import jax
import jax.numpy as jnp
from jax import lax
from jax.experimental import pallas as pl
from jax.experimental.pallas import tpu as pltpu
from jax.experimental.pallas import tpu_sc as plsc

_D = 768
_TOP_K = 64
_TS = 2048
_L = 16
_NCH = _D // _L


def _sc_mask_kernel(pf_hbm, perm_hbm, out_hbm, pfv, permv, bitsv, maskv,
                    redv):
    b = lax.axis_index("s")

    @pl.when((lax.axis_index("c") == 0) & (b < 4))
    def _():
        pltpu.sync_copy(pf_hbm.at[b], pfv)
        pltpu.sync_copy(perm_hbm, permv)

        zero16f = jnp.zeros((_L,), jnp.float32)
        for i in range(_NCH):
            sl = pl.ds(i * _L, _L)
            maskv[sl] = zero16f
            bitsv[sl] = lax.bitcast_convert_type(jnp.abs(pfv[sl]),
                                                 jnp.int32)

        chunks = tuple(bitsv[pl.ds(c * _L, _L)] for c in range(_NCH))

        zero16i = jnp.zeros((_L,), jnp.int32)
        one16i = jnp.ones((_L,), jnp.int32)
        lane = lax.iota(jnp.int32, _L)

        def total_splat(vec):
            for k in (1, 2, 4, 8):
                vec = vec + vec.at[lane ^ k].get(mode="promise_in_bounds")
            return vec

        def count(pred):
            accs = [zero16i, zero16i, zero16i, zero16i]
            for c in range(_NCH):
                accs[c % 4] = accs[c % 4] + jnp.where(pred(c), one16i,
                                                      zero16i)
            return total_splat((accs[0] + accs[1]) + (accs[2] + accs[3]))

        def t_body(i, cur):
            cand = cur | (jnp.int32(1) << (jnp.int32(30) - i))
            cnt = count(lambda c: chunks[c] >= cand)
            return jnp.where(cnt >= _TOP_K, cand, cur)

        t = lax.fori_loop(0, 31, t_body,
                          jnp.zeros((_L,), jnp.int32))

        c_ge = count(lambda c: chunks[c] >= t)

        def idx_of(c):
            return lane + jnp.int32(c * _L)

        def count_sel(jcut):
            return count(lambda c: (chunks[c] > t)
                         | ((chunks[c] == t) & (idx_of(c) <= jcut)))

        def tie_search(_):
            def j_body(i, cur):
                cand = cur | (jnp.int32(1) << (jnp.int32(9) - i))
                return jnp.where(count_sel(cand) < _TOP_K, cand, cur)

            jmax = lax.fori_loop(0, 10, j_body, zero16i)
            return jnp.where(count_sel(zero16i) < _TOP_K, jmax + 1, 0)

        jcut = lax.cond(c_ge[0] == _TOP_K,
                        lambda _: jnp.full((_L,), _D, jnp.int32),
                        tie_search, 0)

        ones16f = jnp.ones((_L,), jnp.float32)
        for c in range(_NCH):
            sel = (chunks[c] > t) | ((chunks[c] == t)
                                     & (idx_of(c) <= jcut))
            plsc.store_scatter(maskv, [permv[pl.ds(c * _L, _L)]],
                               ones16f, mask=sel)

        pltpu.sync_copy(maskv, out_hbm.at[b])


def _sc_mask(pf, perm2):
    mesh = plsc.VectorSubcoreMesh(core_axis_name="c", subcore_axis_name="s",
                                  num_cores=1)
    return pl.kernel(
        _sc_mask_kernel,
        out_type=jax.ShapeDtypeStruct((4, _D), jnp.float32),
        mesh=mesh,
        scratch_types=[
            pltpu.VMEM((_D,), jnp.float32),
            pltpu.VMEM((_D,), jnp.int32),
            pltpu.VMEM((_D,), jnp.int32),
            pltpu.VMEM((_D,), jnp.float32),
            pltpu.VMEM((_L,), jnp.int32),
        ],
        compiler_params=pltpu.CompilerParams(needs_layout_passes=False),
    )(pf, perm2)


def _tc_matmul_kernel(mask_ref, w_ref, nw_ref, x_ref, out_ref, weff_ref):
    b = pl.program_id(0)
    s = pl.program_id(1)

    @pl.when(s == 0)
    def _build_weff():
        weff_ref[...] = (w_ref[...]
                         + mask_ref[pl.ds(b, 1), :] * nw_ref[...]
                         ).astype(jnp.bfloat16)

    x_tile = x_ref[0].astype(jnp.bfloat16)
    out_ref[0] = jax.lax.dot_general(
        x_tile, weff_ref[...], (((1,), (1,)), ((), ())),
        preferred_element_type=jnp.float32)


def kernel(x, W, new_W, perm, prompt_boundary):
    B, S, D = x.shape
    pf = jax.lax.dynamic_index_in_dim(x, prompt_boundary, axis=1,
                                      keepdims=False)
    perm2 = perm.astype(jnp.int32)

    mask = _sc_mask(pf, perm2)

    grid = (B, S // _TS)
    return pl.pallas_call(
        _tc_matmul_kernel,
        grid=grid,
        in_specs=[
            pl.BlockSpec((B, D), lambda b, s: (0, 0)),
            pl.BlockSpec((D, D), lambda b, s: (0, 0)),
            pl.BlockSpec((D, D), lambda b, s: (0, 0)),
            pl.BlockSpec((1, _TS, D), lambda b, s: (b, s, 0)),
        ],
        out_specs=pl.BlockSpec((1, _TS, D), lambda b, s: (b, s, 0)),
        out_shape=jax.ShapeDtypeStruct((B, S, D), jnp.float32),
        scratch_shapes=[pltpu.VMEM((_D, _D), jnp.bfloat16)],
        compiler_params=pltpu.CompilerParams(
            dimension_semantics=("arbitrary", "arbitrary")),
    )(mask, W, new_W, x)


# speedup vs baseline: 1.0267x; 1.0020x over previous
import jax
import jax.numpy as jnp
from jax import lax
from jax.experimental import pallas as pl
from jax.experimental.pallas import tpu as pltpu
from jax.experimental.pallas import tpu_sc as plsc

_D = 768
_TOP_K = 64
_TS = 2048
_L = 16
_NCH = _D // _L


def _sc_mask_kernel(pf_hbm, perm_hbm, out_hbm, pfv, permv, bitsv, maskv):
    b = lax.axis_index("s")

    @pl.when((lax.axis_index("c") == 0) & (b < 4))
    def _():
        pltpu.sync_copy(pf_hbm.at[b], pfv)
        pltpu.sync_copy(perm_hbm, permv)

        zero16f = jnp.zeros((_L,), jnp.float32)
        for i in range(_NCH):
            sl = pl.ds(i * _L, _L)
            maskv[sl] = zero16f
            bitsv[sl] = lax.bitcast_convert_type(jnp.abs(pfv[sl]),
                                                 jnp.int32)

        chunks = tuple(bitsv[pl.ds(c * _L, _L)] for c in range(_NCH))

        zero16i = jnp.zeros((_L,), jnp.int32)
        one16i = jnp.ones((_L,), jnp.int32)
        lane = lax.iota(jnp.int32, _L)

        def total_splat(vec):
            for k in (1, 2, 4, 8):
                vec = vec + vec.at[lane ^ k].get(mode="promise_in_bounds")
            return vec

        def count(pred):
            accs = [zero16i, zero16i, zero16i, zero16i]
            for c in range(_NCH):
                accs[c % 4] = accs[c % 4] + jnp.where(pred(c), one16i,
                                                      zero16i)
            return total_splat((accs[0] + accs[1]) + (accs[2] + accs[3]))

        def t_body(i, cur):
            cand = cur | (jnp.int32(1) << (jnp.int32(30) - i))
            cnt = count(lambda c: chunks[c] >= cand)
            return jnp.where(cnt >= _TOP_K, cand, cur)

        t = lax.fori_loop(0, 31, t_body,
                          jnp.zeros((_L,), jnp.int32))

        c_ge = count(lambda c: chunks[c] >= t)

        def idx_of(c):
            return lane + jnp.int32(c * _L)

        def count_sel(jcut):
            return count(lambda c: (chunks[c] > t)
                         | ((chunks[c] == t) & (idx_of(c) <= jcut)))

        def tie_search(_):
            def j_body(i, cur):
                cand = cur | (jnp.int32(1) << (jnp.int32(9) - i))
                return jnp.where(count_sel(cand) < _TOP_K, cand, cur)

            jmax = lax.fori_loop(0, 10, j_body, zero16i)
            return jnp.where(count_sel(zero16i) < _TOP_K, jmax + 1, 0)

        jcut = lax.cond(c_ge[0] == _TOP_K,
                        lambda _: jnp.full((_L,), _D, jnp.int32),
                        tie_search, 0)

        ones16f = jnp.ones((_L,), jnp.float32)
        for c in range(_NCH):
            sel = (chunks[c] > t) | ((chunks[c] == t)
                                     & (idx_of(c) <= jcut))
            plsc.store_scatter(maskv, [permv[pl.ds(c * _L, _L)]],
                               ones16f, mask=sel)

        pltpu.sync_copy(maskv, out_hbm.at[b])


def _sc_mask(pf, perm2):
    mesh = plsc.VectorSubcoreMesh(core_axis_name="c", subcore_axis_name="s",
                                  num_cores=1)
    return pl.kernel(
        _sc_mask_kernel,
        out_type=jax.ShapeDtypeStruct((4, _D), jnp.float32),
        mesh=mesh,
        scratch_types=[
            pltpu.VMEM((_D,), jnp.float32),
            pltpu.VMEM((_D,), jnp.int32),
            pltpu.VMEM((_D,), jnp.int32),
            pltpu.VMEM((_D,), jnp.float32),
        ],
        compiler_params=pltpu.CompilerParams(needs_layout_passes=False),
    )(pf, perm2)


def _tc_matmul_kernel(mask_ref, w_ref, nw_ref, x_ref, out_ref, weff_ref):
    b = pl.program_id(0)
    s = pl.program_id(1)

    @pl.when(s == 0)
    def _build_weff():
        weff_ref[...] = (w_ref[...]
                         + mask_ref[pl.ds(b, 1), :] * nw_ref[...]
                         ).astype(jnp.bfloat16)

    x_tile = x_ref[0].astype(jnp.bfloat16)
    out_ref[0] = jax.lax.dot_general(
        x_tile, weff_ref[...], (((1,), (1,)), ((), ())),
        preferred_element_type=jnp.float32)


def kernel(x, W, new_W, perm, prompt_boundary):
    B, S, D = x.shape
    pf = jax.lax.dynamic_index_in_dim(x, prompt_boundary, axis=1,
                                      keepdims=False)
    perm2 = perm.astype(jnp.int32)

    mask = _sc_mask(pf, perm2)

    grid = (B, S // _TS)
    return pl.pallas_call(
        _tc_matmul_kernel,
        grid=grid,
        in_specs=[
            pl.BlockSpec((B, D), lambda b, s: (0, 0)),
            pl.BlockSpec((D, D), lambda b, s: (0, 0)),
            pl.BlockSpec((D, D), lambda b, s: (0, 0)),
            pl.BlockSpec((1, _TS, D), lambda b, s: (b, s, 0)),
        ],
        out_specs=pl.BlockSpec((1, _TS, D), lambda b, s: (b, s, 0)),
        out_shape=jax.ShapeDtypeStruct((B, S, D), jnp.float32),
        scratch_shapes=[pltpu.VMEM((_D, _D), jnp.bfloat16)],
        compiler_params=pltpu.CompilerParams(
            dimension_semantics=("arbitrary", "arbitrary")),
    )(mask, W, new_W, x)
